# baseline (speedup 1.0000x reference)
import functools

import jax
import jax.numpy as jnp
from jax import lax
from jax.experimental import pallas as pl
from jax.experimental.pallas import tpu as pltpu
from jax.experimental.pallas import tpu_sc as plsc

B = 4096
D = 64
L = 16
NC = 2
NS = 16
NW = NC * NS
BPW = B // NW
V = 100000
CB = 3584
NB = 14
VPH = NB * CB


def _pack_body(lo_ref, hi_ref, o_ref):
    eye = (lax.broadcasted_iota(jnp.int32, (D, D), 0)
           == lax.broadcasted_iota(jnp.int32, (D, D), 1)).astype(jnp.bfloat16)
    dn = (((0,), (0,)), ((), ()))
    lo_t = lax.dot_general(lo_ref[...].astype(jnp.bfloat16), eye, dn,
                           preferred_element_type=jnp.float32)
    hi_t = lax.dot_general(hi_ref[...].astype(jnp.bfloat16), eye, dn,
                           preferred_element_type=jnp.float32)
    o_ref[...] = jnp.concatenate([lo_t, hi_t], axis=1)


@jax.jit
def _pack_table(t):
    return pl.pallas_call(
        _pack_body,
        grid=(NB,),
        in_specs=[
            pl.BlockSpec((D, CB), lambda i: (0, i)),
            pl.BlockSpec((D, CB), lambda i: (0, i + NB)),
        ],
        out_specs=pl.BlockSpec((CB, 2 * D), lambda i: (i, 0)),
        out_shape=jax.ShapeDtypeStruct((VPH, 2 * D), jnp.float32),
        compiler_params=pltpu.CompilerParams(
            fuse_transposed_lhs_in_matmul=True),
    )(t, t)


def _sc_g_body(pi_hbm, pidx_hbm, nidx_hbm, out_hbm,
               praw_v, nraw_v, pidx_v, nidx_v,
               prows_v, nrows_v, gp_v, sem_p, sem_n):
    wid = lax.axis_index("s") * NC + lax.axis_index("c")
    base = wid * BPW
    pltpu.sync_copy(pidx_hbm.at[pl.ds(base, BPW)], praw_v)
    pltpu.sync_copy(nidx_hbm.at[pl.ds(base, BPW)], nraw_v)
    for c in range(BPW // L):
        s = pl.ds(c * L, L)
        pidx_v[s] = jnp.where(praw_v[s] >= VPH, praw_v[s] - VPH, praw_v[s])
        nidx_v[s] = jnp.where(nraw_v[s] >= VPH, nraw_v[s] - VPH, nraw_v[s])
    cp = pltpu.async_copy(pi_hbm.at[pidx_v], prows_v, sem_p)
    cn = pltpu.async_copy(pi_hbm.at[nidx_v], nrows_v, sem_n)
    cp.wait()
    cn.wait()

    iota = lax.iota(jnp.int32, L)
    skew = (iota * 9) & (D - 1)
    gbase = (iota & 1) * D + skew

    def group(g, carry):
        s = pl.ds(g * L, L)
        rows = g * L + iota
        grows = rows >> 1
        zero = jnp.zeros((L,), jnp.int32)
        dvec = jnp.full((L,), D, jnp.int32)
        pb = jnp.where(praw_v[s] >= VPH, dvec, zero) + skew
        nb = jnp.where(nraw_v[s] >= VPH, dvec, zero) + skew
        for k in range(D):
            kv = jnp.full((L,), k, jnp.int32)
            coff = ((skew + kv) & (D - 1)) - skew
            pv = plsc.load_gather(prows_v, [rows, pb + coff])
            nv = plsc.load_gather(nrows_v, [rows, nb + coff])
            plsc.store_scatter(gp_v, [grows, gbase + coff], nv - pv)
        return carry

    lax.fori_loop(0, BPW // L, group, 0)
    gb2 = pl.multiple_of(base // 2, 8)
    pltpu.sync_copy(gp_v, out_hbm.at[pl.ds(gb2, BPW // 2)])


@jax.jit
def _sc_g(pi, pidx, nidx):
    mesh = plsc.VectorSubcoreMesh(core_axis_name="c", subcore_axis_name="s")
    return pl.kernel(
        _sc_g_body,
        out_type=jax.ShapeDtypeStruct((B // 2, 2 * D), jnp.float32),
        mesh=mesh,
        scratch_types=[
            pltpu.VMEM((BPW,), jnp.int32),
            pltpu.VMEM((BPW,), jnp.int32),
            pltpu.VMEM((BPW,), jnp.int32),
            pltpu.VMEM((BPW,), jnp.int32),
            pltpu.VMEM((BPW, 2 * D), jnp.float32),
            pltpu.VMEM((BPW, 2 * D), jnp.float32),
            pltpu.VMEM((BPW // 2, 2 * D), jnp.float32),
            pltpu.SemaphoreType.DMA,
            pltpu.SemaphoreType.DMA,
        ],
        compiler_params=pltpu.CompilerParams(
            needs_layout_passes=False, use_tc_tiling_on_sc=True),
    )(pi, pidx, nidx)


def _sc_dot_body(pu_hbm, uidx_hbm, gp_hbm, out_hbm,
                 uraw_v, uidx_v, urows_v, gp_v, diffs_v, sem_u, sem_g):
    wid = lax.axis_index("s") * NC + lax.axis_index("c")
    base = wid * BPW
    pltpu.sync_copy(uidx_hbm.at[pl.ds(base, BPW)], uraw_v)
    for c in range(BPW // L):
        s = pl.ds(c * L, L)
        uidx_v[s] = jnp.where(uraw_v[s] >= VPH, uraw_v[s] - VPH, uraw_v[s])
    cu = pltpu.async_copy(pu_hbm.at[uidx_v], urows_v, sem_u)
    gb2 = pl.multiple_of(base // 2, 8)
    cg = pltpu.async_copy(gp_hbm.at[pl.ds(gb2, BPW // 2)], gp_v, sem_g)
    cu.wait()
    cg.wait()

    iota = lax.iota(jnp.int32, L)
    skew = (iota * 9) & (D - 1)
    gbase = (iota & 1) * D + skew

    def group(g, carry):
        s = pl.ds(g * L, L)
        rows = g * L + iota
        grows = rows >> 1
        zero = jnp.zeros((L,), jnp.int32)
        dvec = jnp.full((L,), D, jnp.int32)
        ub = jnp.where(uraw_v[s] >= VPH, dvec, zero) + skew
        acc = jnp.zeros((L,), jnp.float32)
        for k in range(D):
            kv = jnp.full((L,), k, jnp.int32)
            coff = ((skew + kv) & (D - 1)) - skew
            u = plsc.load_gather(urows_v, [rows, ub + coff])
            gv = plsc.load_gather(gp_v, [grows, gbase + coff])
            acc = acc + u * gv
        diffs_v[s] = acc
        return carry

    lax.fori_loop(0, BPW // L, group, 0)
    pltpu.sync_copy(diffs_v, out_hbm.at[pl.ds(base, BPW)])


@jax.jit
def _sc_dot(pu, uidx, gp):
    mesh = plsc.VectorSubcoreMesh(core_axis_name="c", subcore_axis_name="s")
    return pl.kernel(
        _sc_dot_body,
        out_type=jax.ShapeDtypeStruct((B,), jnp.float32),
        mesh=mesh,
        scratch_types=[
            pltpu.VMEM((BPW,), jnp.int32),
            pltpu.VMEM((BPW,), jnp.int32),
            pltpu.VMEM((BPW, 2 * D), jnp.float32),
            pltpu.VMEM((BPW // 2, 2 * D), jnp.float32),
            pltpu.VMEM((BPW,), jnp.float32),
            pltpu.SemaphoreType.DMA,
            pltpu.SemaphoreType.DMA,
        ],
        compiler_params=pltpu.CompilerParams(
            needs_layout_passes=False, use_tc_tiling_on_sc=True),
    )(pu, uidx, gp)


def _tc_loss_body(x_ref, o_ref):
    x = x_ref[...]
    sp = jnp.maximum(x, 0.0) + jnp.log1p(jnp.exp(-jnp.abs(x)))
    o_ref[0, 0] = jnp.sum(sp) * (1.0 / B)


@jax.jit
def _tc_loss(diffs2d):
    out = pl.pallas_call(
        _tc_loss_body,
        out_shape=jax.ShapeDtypeStruct((1, 1), jnp.float32),
        out_specs=pl.BlockSpec(memory_space=pltpu.SMEM),
    )(diffs2d)
    return out[0, 0]


def kernel(users, items, users_feature, items_feature):
    uidx = users.reshape(B)
    pidx = items[:, 0]
    nidx = items[:, 1]
    pi = _pack_table(items_feature.T)
    gp = _sc_g(pi, pidx, nidx)
    pu = _pack_table(users_feature.T)
    diffs = _sc_dot(pu, uidx, gp)
    return _tc_loss(diffs.reshape(NW, BPW))


# speedup vs baseline: 1.1293x; 1.1293x over previous
import functools

import jax
import jax.numpy as jnp
from jax import lax
from jax.experimental import pallas as pl
from jax.experimental.pallas import tpu as pltpu
from jax.experimental.pallas import tpu_sc as plsc

B = 4096
D = 64
L = 16
NC = 2
NS = 16
NW = NC * NS
BPW = B // NW
V = 100000
CB = 12544
NB = 4
VPH = NB * CB


def _pack_body(lo_ref, hi_ref, o_ref):
    eye = (lax.broadcasted_iota(jnp.int32, (D, D), 0)
           == lax.broadcasted_iota(jnp.int32, (D, D), 1)).astype(jnp.bfloat16)
    dn = (((0,), (0,)), ((), ()))
    lo_t = lax.dot_general(lo_ref[...].astype(jnp.bfloat16), eye, dn,
                           preferred_element_type=jnp.float32)
    hi_t = lax.dot_general(hi_ref[...].astype(jnp.bfloat16), eye, dn,
                           preferred_element_type=jnp.float32)
    o_ref[...] = jnp.concatenate([lo_t, hi_t], axis=1)


@jax.jit
def _pack_table(t):
    return pl.pallas_call(
        _pack_body,
        grid=(NB,),
        in_specs=[
            pl.BlockSpec((D, CB), lambda i: (0, i)),
            pl.BlockSpec((D, CB), lambda i: (0, i + NB)),
        ],
        out_specs=pl.BlockSpec((CB, 2 * D), lambda i: (i, 0)),
        out_shape=jax.ShapeDtypeStruct((VPH, 2 * D), jnp.float32),
        compiler_params=pltpu.CompilerParams(
            fuse_transposed_lhs_in_matmul=True),
    )(t, t)


def _sc_g_body(pi_hbm, pidx_hbm, nidx_hbm, out_hbm,
               praw_v, nraw_v, pidx_v, nidx_v,
               prows_v, nrows_v, gp_v, sem_p, sem_n):
    wid = lax.axis_index("s") * NC + lax.axis_index("c")
    base = wid * BPW
    pltpu.sync_copy(pidx_hbm.at[pl.ds(base, BPW)], praw_v)
    pltpu.sync_copy(nidx_hbm.at[pl.ds(base, BPW)], nraw_v)
    for c in range(BPW // L):
        s = pl.ds(c * L, L)
        pidx_v[s] = jnp.where(praw_v[s] >= VPH, praw_v[s] - VPH, praw_v[s])
        nidx_v[s] = jnp.where(nraw_v[s] >= VPH, nraw_v[s] - VPH, nraw_v[s])
    cp = pltpu.async_copy(pi_hbm.at[pidx_v], prows_v, sem_p)
    cn = pltpu.async_copy(pi_hbm.at[nidx_v], nrows_v, sem_n)
    cp.wait()
    cn.wait()

    iota = lax.iota(jnp.int32, L)
    skew = (iota * 9) & (D - 1)
    gbase = (iota & 1) * D + skew

    def group(g, carry):
        s = pl.ds(g * L, L)
        rows = g * L + iota
        grows = rows >> 1
        zero = jnp.zeros((L,), jnp.int32)
        dvec = jnp.full((L,), D, jnp.int32)
        pb = jnp.where(praw_v[s] >= VPH, dvec, zero) + skew
        nb = jnp.where(nraw_v[s] >= VPH, dvec, zero) + skew
        for k in range(D):
            kv = jnp.full((L,), k, jnp.int32)
            coff = ((skew + kv) & (D - 1)) - skew
            pv = plsc.load_gather(prows_v, [rows, pb + coff])
            nv = plsc.load_gather(nrows_v, [rows, nb + coff])
            plsc.store_scatter(gp_v, [grows, gbase + coff], nv - pv)
        return carry

    lax.fori_loop(0, BPW // L, group, 0)
    gb2 = pl.multiple_of(base // 2, 8)
    pltpu.sync_copy(gp_v, out_hbm.at[pl.ds(gb2, BPW // 2)])


@jax.jit
def _sc_g(pi, pidx, nidx):
    mesh = plsc.VectorSubcoreMesh(core_axis_name="c", subcore_axis_name="s")
    return pl.kernel(
        _sc_g_body,
        out_type=jax.ShapeDtypeStruct((B // 2, 2 * D), jnp.float32),
        mesh=mesh,
        scratch_types=[
            pltpu.VMEM((BPW,), jnp.int32),
            pltpu.VMEM((BPW,), jnp.int32),
            pltpu.VMEM((BPW,), jnp.int32),
            pltpu.VMEM((BPW,), jnp.int32),
            pltpu.VMEM((BPW, 2 * D), jnp.float32),
            pltpu.VMEM((BPW, 2 * D), jnp.float32),
            pltpu.VMEM((BPW // 2, 2 * D), jnp.float32),
            pltpu.SemaphoreType.DMA,
            pltpu.SemaphoreType.DMA,
        ],
        compiler_params=pltpu.CompilerParams(
            needs_layout_passes=False, use_tc_tiling_on_sc=True),
    )(pi, pidx, nidx)


def _sc_dot_body(pu_hbm, uidx_hbm, gp_hbm, out_hbm,
                 uraw_v, uidx_v, urows_v, gp_v, diffs_v, sem_u, sem_g):
    wid = lax.axis_index("s") * NC + lax.axis_index("c")
    base = wid * BPW
    pltpu.sync_copy(uidx_hbm.at[pl.ds(base, BPW)], uraw_v)
    for c in range(BPW // L):
        s = pl.ds(c * L, L)
        uidx_v[s] = jnp.where(uraw_v[s] >= VPH, uraw_v[s] - VPH, uraw_v[s])
    cu = pltpu.async_copy(pu_hbm.at[uidx_v], urows_v, sem_u)
    gb2 = pl.multiple_of(base // 2, 8)
    cg = pltpu.async_copy(gp_hbm.at[pl.ds(gb2, BPW // 2)], gp_v, sem_g)
    cu.wait()
    cg.wait()

    iota = lax.iota(jnp.int32, L)
    skew = (iota * 9) & (D - 1)
    gbase = (iota & 1) * D + skew

    def group(g, carry):
        s = pl.ds(g * L, L)
        rows = g * L + iota
        grows = rows >> 1
        zero = jnp.zeros((L,), jnp.int32)
        dvec = jnp.full((L,), D, jnp.int32)
        ub = jnp.where(uraw_v[s] >= VPH, dvec, zero) + skew
        acc = jnp.zeros((L,), jnp.float32)
        for k in range(D):
            kv = jnp.full((L,), k, jnp.int32)
            coff = ((skew + kv) & (D - 1)) - skew
            u = plsc.load_gather(urows_v, [rows, ub + coff])
            gv = plsc.load_gather(gp_v, [grows, gbase + coff])
            acc = acc + u * gv
        diffs_v[s] = acc
        return carry

    lax.fori_loop(0, BPW // L, group, 0)
    pltpu.sync_copy(diffs_v, out_hbm.at[pl.ds(base, BPW)])


@jax.jit
def _sc_dot(pu, uidx, gp):
    mesh = plsc.VectorSubcoreMesh(core_axis_name="c", subcore_axis_name="s")
    return pl.kernel(
        _sc_dot_body,
        out_type=jax.ShapeDtypeStruct((B,), jnp.float32),
        mesh=mesh,
        scratch_types=[
            pltpu.VMEM((BPW,), jnp.int32),
            pltpu.VMEM((BPW,), jnp.int32),
            pltpu.VMEM((BPW, 2 * D), jnp.float32),
            pltpu.VMEM((BPW // 2, 2 * D), jnp.float32),
            pltpu.VMEM((BPW,), jnp.float32),
            pltpu.SemaphoreType.DMA,
            pltpu.SemaphoreType.DMA,
        ],
        compiler_params=pltpu.CompilerParams(
            needs_layout_passes=False, use_tc_tiling_on_sc=True),
    )(pu, uidx, gp)


def _tc_loss_body(x_ref, o_ref):
    x = x_ref[...]
    sp = jnp.maximum(x, 0.0) + jnp.log1p(jnp.exp(-jnp.abs(x)))
    o_ref[0, 0] = jnp.sum(sp) * (1.0 / B)


@jax.jit
def _tc_loss(diffs2d):
    out = pl.pallas_call(
        _tc_loss_body,
        out_shape=jax.ShapeDtypeStruct((1, 1), jnp.float32),
        out_specs=pl.BlockSpec(memory_space=pltpu.SMEM),
    )(diffs2d)
    return out[0, 0]


def kernel(users, items, users_feature, items_feature):
    uidx = users.reshape(B)
    pidx = items[:, 0]
    nidx = items[:, 1]
    pi = _pack_table(items_feature.T)
    gp = _sc_g(pi, pidx, nidx)
    pu = _pack_table(users_feature.T)
    diffs = _sc_dot(pu, uidx, gp)
    return _tc_loss(diffs.reshape(NW, BPW))
